# Initial kernel scaffold; baseline (speedup 1.0000x reference)
import functools

import jax
import jax.numpy as jnp
from jax.experimental import pallas as pl
from jax.experimental.pallas import tpu as pltpu

DIMF = 256
N = 1024
K = 32
MD = 16
NTOK = 21
T = 256
NT = N // T
EH = 2 * DIMF + 1
H1 = EH * 2


def _sig(t):
    return 1.0 / (1.0 + jnp.exp(-t))


def _silu(t):
    return t * _sig(t)


def _tanh(t):
    return 1.0 - 2.0 / (jnp.exp(2.0 * t) + 1.0)


def _gelu(t):
    c = 0.7978845608028654
    return 0.5 * t * (1.0 + _tanh(c * (t + 0.044715 * (t * t * t))))


def _emb_kernel(zf_ref, tok_ref, pos_ref, out_ref):
    zf = zf_ref[0]
    iot = jax.lax.broadcasted_iota(jnp.int32, (1, NTOK), 1).astype(jnp.float32)
    oh = (zf == iot).astype(jnp.float32)
    out_ref[0] = (
        jnp.dot(oh, tok_ref[...], preferred_element_type=jnp.float32)
        + pos_ref[...]
    )


def _layer_kernel(
    xt_ref, xc_ref, f_ref,
    wi_ref, wj_ref, wd_ref, b1_ref, w2_ref, b2_ref,
    wc1_ref, bc1_ref, wc2_ref, bc2_ref, sc_ref,
    wn1_ref, bn1_ref, wn2_ref, bn2_ref,
    fout_ref, cout_ref,
    d_ref, pi_ref, mi_ref, cd_ref,
):
    t = pl.program_id(1)
    fi = f_ref[0, pl.ds(t * T, T), :]
    xi = xc_ref[0, pl.ds(t * T, T), :]
    xrow = xt_ref[0]

    d = None
    for c in range(3):
        r = xi[:, c:c + 1] - xrow[c:c + 1, :]
        r2 = r * r
        d = r2 if d is None else d + r2
    d_ref[...] = d

    pi_ref[...] = (
        jnp.dot(fi, wi_ref[...], preferred_element_type=jnp.float32)
        + b1_ref[...]
    )
    mi_ref[...] = jnp.zeros((T, MD), jnp.float32)
    cd_ref[...] = jnp.zeros((T, 8), jnp.float32)

    iota = jax.lax.broadcasted_iota(jnp.int32, (T, N), 1)

    def body(r, carry):
        del r
        dd = d_ref[...]
        v = jnp.min(dd, axis=1, keepdims=True)
        cand = jnp.where(dd == v, iota, N)
        jmin = jnp.min(cand, axis=1, keepdims=True)
        sel = cand == jmin
        d_ref[...] = jnp.where(sel, jnp.inf, dd)
        oh = sel.astype(jnp.float32)
        g = jnp.dot(oh, f_ref[0], preferred_element_type=jnp.float32)
        xj = jnp.dot(oh, xc_ref[0], preferred_element_type=jnp.float32)
        h1 = (
            jnp.dot(g, wj_ref[...], preferred_element_type=jnp.float32)
            + pi_ref[...]
            + v * wd_ref[...]
        )
        a1 = _silu(h1)
        m = _silu(jnp.dot(a1, w2_ref[...], preferred_element_type=jnp.float32)
                  + b2_ref[...])
        c1 = _silu(jnp.dot(m, wc1_ref[...], preferred_element_type=jnp.float32)
                   + bc1_ref[...])
        wgt = (jnp.dot(c1, wc2_ref[...], preferred_element_type=jnp.float32)
               + bc2_ref[...])
        rel = xi - xj
        nrm = jnp.sqrt(jnp.sum(rel * rel, axis=1, keepdims=True))
        reln = rel / jnp.maximum(nrm, 1e-8) * sc_ref[...]
        cd_ref[...] += wgt * reln
        mi_ref[...] += m
        return carry

    jax.lax.fori_loop(0, K, body, 0)

    ni = jnp.concatenate([fi, mi_ref[...]], axis=-1)
    n1 = _silu(jnp.dot(ni, wn1_ref[...], preferred_element_type=jnp.float32)
               + bn1_ref[...])
    node = (jnp.dot(n1, wn2_ref[...], preferred_element_type=jnp.float32)
            + bn2_ref[...])
    fout_ref[0] = fi + node
    cout_ref[0] = xi + cd_ref[...]


def _ffn_kernel(f_ref, g1_ref, bb1_ref, wf1_ref, bf1_ref, wf2_ref, bf2_ref,
                g2_ref, bb2_ref, out_ref):
    h = f_ref[0]
    hh = h + h
    mu = jnp.mean(hh, axis=-1, keepdims=True)
    var = jnp.mean((hh - mu) ** 2, axis=-1, keepdims=True)
    hn = (hh - mu) / jnp.sqrt(var + 1e-5) * g1_ref[...] + bb1_ref[...]
    t1 = _gelu(jnp.dot(hn, wf1_ref[...], preferred_element_type=jnp.float32)
               + bf1_ref[...])
    h2 = (jnp.dot(t1, wf2_ref[...], preferred_element_type=jnp.float32)
          + bf2_ref[...])
    s = hn + h2
    mu2 = jnp.mean(s, axis=-1, keepdims=True)
    var2 = jnp.mean((s - mu2) ** 2, axis=-1, keepdims=True)
    out_ref[0] = (s - mu2) / jnp.sqrt(var2 + 1e-5) * g2_ref[...] + bb2_ref[...]


def _full(shape):
    return pl.BlockSpec(shape, lambda b, t: (0,) * len(shape))


@jax.jit
def kernel(z, x, params):
    B = z.shape[0]
    f32 = jnp.float32

    zf = z.astype(f32)[..., None]
    xc = jnp.concatenate(
        [x, jnp.zeros((B, N, 5), f32)], axis=-1)

    tok = params["token_emb"]
    pos = params["pos_emb"]

    feats = pl.pallas_call(
        _emb_kernel,
        grid=(B, NT),
        in_specs=[
            pl.BlockSpec((1, T, 1), lambda b, t: (b, t, 0)),
            _full((NTOK, DIMF)),
            pl.BlockSpec((T, DIMF), lambda b, t: (t, 0)),
        ],
        out_specs=pl.BlockSpec((1, T, DIMF), lambda b, t: (b, t, 0)),
        out_shape=jax.ShapeDtypeStruct((B, N, DIMF), f32),
    )(zf, tok, pos)

    coors = xc
    for lp in params["layers"]:
        w1 = lp["edge1"]["w"]
        args = (
            jnp.transpose(coors, (0, 2, 1)),
            coors,
            feats,
            w1[:DIMF], w1[DIMF:2 * DIMF], w1[2 * DIMF:2 * DIMF + 1],
            lp["edge1"]["b"][None],
            lp["edge2"]["w"], lp["edge2"]["b"][None],
            lp["coor1"]["w"], lp["coor1"]["b"][None],
            lp["coor2"]["w"], lp["coor2"]["b"][None],
            lp["coors_scale"].reshape(1, 1),
            lp["node1"]["w"], lp["node1"]["b"][None],
            lp["node2"]["w"], lp["node2"]["b"][None],
        )
        feats, coors = pl.pallas_call(
            _layer_kernel,
            grid=(B, NT),
            in_specs=[
                pl.BlockSpec((1, 8, N), lambda b, t: (b, 0, 0)),
                pl.BlockSpec((1, N, 8), lambda b, t: (b, 0, 0)),
                pl.BlockSpec((1, N, DIMF), lambda b, t: (b, 0, 0)),
                _full((DIMF, H1)), _full((DIMF, H1)), _full((1, H1)),
                _full((1, H1)), _full((H1, MD)), _full((1, MD)),
                _full((MD, 4 * MD)), _full((1, 4 * MD)),
                _full((4 * MD, 1)), _full((1, 1)), _full((1, 1)),
                _full((DIMF + MD, 2 * DIMF)), _full((1, 2 * DIMF)),
                _full((2 * DIMF, DIMF)), _full((1, DIMF)),
            ],
            out_specs=[
                pl.BlockSpec((1, T, DIMF), lambda b, t: (b, t, 0)),
                pl.BlockSpec((1, T, 8), lambda b, t: (b, t, 0)),
            ],
            out_shape=[
                jax.ShapeDtypeStruct((B, N, DIMF), f32),
                jax.ShapeDtypeStruct((B, N, 8), f32),
            ],
            scratch_shapes=[
                pltpu.VMEM((T, N), f32),
                pltpu.VMEM((T, H1), f32),
                pltpu.VMEM((T, MD), f32),
                pltpu.VMEM((T, 8), f32),
            ],
        )(*args)

    h = pl.pallas_call(
        _ffn_kernel,
        grid=(B, NT),
        in_specs=[
            pl.BlockSpec((1, T, DIMF), lambda b, t: (b, t, 0)),
            _full((1, DIMF)), _full((1, DIMF)),
            _full((DIMF, 4 * DIMF)), _full((1, 4 * DIMF)),
            _full((4 * DIMF, DIMF)), _full((1, DIMF)),
            _full((1, DIMF)), _full((1, DIMF)),
        ],
        out_specs=pl.BlockSpec((1, T, DIMF), lambda b, t: (b, t, 0)),
        out_shape=jax.ShapeDtypeStruct((B, N, DIMF), f32),
    )(feats, params["norm1_g"][None], params["norm1_b"][None],
      params["ffn1"]["w"], params["ffn1"]["b"][None],
      params["ffn2"]["w"], params["ffn2"]["b"][None],
      params["norm2_g"][None], params["norm2_b"][None])

    return h, coors[..., :3]


# speedup vs baseline: 3.1458x; 3.1458x over previous
import functools

import jax
import jax.numpy as jnp
from jax.experimental import pallas as pl
from jax.experimental.pallas import tpu as pltpu

DIMF = 256
N = 1024
K = 32
MD = 16
NTOK = 21
T = 256
NT = N // T
EH = 2 * DIMF + 1
H1 = EH * 2


def _silu(t):
    return t * jax.nn.sigmoid(t)


def _gelu(t):
    return jax.nn.gelu(t)


def _emb_kernel(zf_ref, tok_ref, pos_ref, out_ref):
    zf = zf_ref[0]
    iot = jax.lax.broadcasted_iota(jnp.int32, (1, NTOK), 1).astype(jnp.float32)
    oh = (zf == iot).astype(jnp.float32)
    out_ref[0] = (
        jnp.dot(oh, tok_ref[...], preferred_element_type=jnp.float32, precision=jax.lax.Precision.HIGHEST)
        + pos_ref[...]
    )


def _layer_kernel(
    xt_ref, xc_ref, f_ref,
    wi_ref, wj_ref, wd_ref, b1_ref, w2_ref, b2_ref,
    wc1_ref, bc1_ref, wc2_ref, bc2_ref, sc_ref,
    wn1_ref, bn1_ref, wn2_ref, bn2_ref,
    fout_ref, cout_ref,
    d_ref, pi_ref, mi_ref, cd_ref,
):
    t = pl.program_id(1)
    fi = f_ref[0, pl.ds(t * T, T), :]
    xi = xc_ref[0, pl.ds(t * T, T), :]
    xrow = xt_ref[0]

    d = None
    for c in range(3):
        r = xi[:, c:c + 1] - xrow[c:c + 1, :]
        r2 = r * r
        d = r2 if d is None else d + r2
    d_ref[...] = d

    pi_ref[...] = (
        jnp.dot(fi, wi_ref[...], preferred_element_type=jnp.float32)
        + b1_ref[...]
    )
    mi_ref[...] = jnp.zeros((T, MD), jnp.float32)
    cd_ref[...] = jnp.zeros((T, 8), jnp.float32)

    iota = jax.lax.broadcasted_iota(jnp.int32, (T, N), 1)

    def body(r, carry):
        del r
        dd = d_ref[...]
        v = jnp.min(dd, axis=1, keepdims=True)
        cand = jnp.where(dd == v, iota, N)
        jmin = jnp.min(cand, axis=1, keepdims=True)
        sel = cand == jmin
        d_ref[...] = jnp.where(sel, jnp.inf, dd)
        oh = sel.astype(jnp.float32)
        g = jnp.dot(oh, f_ref[0], preferred_element_type=jnp.float32, precision=jax.lax.Precision.HIGHEST)
        xj = jnp.dot(oh, xc_ref[0], preferred_element_type=jnp.float32, precision=jax.lax.Precision.HIGHEST)
        h1 = (
            jnp.dot(g, wj_ref[...], preferred_element_type=jnp.float32)
            + pi_ref[...]
            + v.astype(jnp.bfloat16).astype(jnp.float32)
            * wd_ref[...].astype(jnp.bfloat16).astype(jnp.float32)
        )
        a1 = _silu(h1)
        m = _silu(jnp.dot(a1, w2_ref[...], preferred_element_type=jnp.float32)
                  + b2_ref[...])
        c1 = _silu(jnp.dot(m, wc1_ref[...], preferred_element_type=jnp.float32)
                   + bc1_ref[...])
        wgt = (jnp.dot(c1, wc2_ref[...], preferred_element_type=jnp.float32)
               + bc2_ref[...])
        rel = xi - xj
        nrm = jnp.sqrt(jnp.sum(rel * rel, axis=1, keepdims=True))
        reln = rel / jnp.maximum(nrm, 1e-8) * sc_ref[...]
        cd_ref[...] += wgt * reln
        mi_ref[...] += m
        return carry

    jax.lax.fori_loop(0, K, body, 0)

    ni = jnp.concatenate([fi, mi_ref[...]], axis=-1)
    n1 = _silu(jnp.dot(ni, wn1_ref[...], preferred_element_type=jnp.float32)
               + bn1_ref[...])
    node = (jnp.dot(n1, wn2_ref[...], preferred_element_type=jnp.float32)
            + bn2_ref[...])
    fout_ref[0] = fi + node
    cout_ref[0] = xi + cd_ref[...]


def _ffn_kernel(f_ref, g1_ref, bb1_ref, wf1_ref, bf1_ref, wf2_ref, bf2_ref,
                g2_ref, bb2_ref, out_ref):
    h = f_ref[0]
    hh = h + h
    mu = jnp.mean(hh, axis=-1, keepdims=True)
    var = jnp.mean((hh - mu) ** 2, axis=-1, keepdims=True)
    hn = (hh - mu) / jnp.sqrt(var + 1e-5) * g1_ref[...] + bb1_ref[...]
    t1 = _gelu(jnp.dot(hn, wf1_ref[...], preferred_element_type=jnp.float32)
               + bf1_ref[...])
    h2 = (jnp.dot(t1, wf2_ref[...], preferred_element_type=jnp.float32)
          + bf2_ref[...])
    s = hn + h2
    mu2 = jnp.mean(s, axis=-1, keepdims=True)
    var2 = jnp.mean((s - mu2) ** 2, axis=-1, keepdims=True)
    out_ref[0] = (s - mu2) / jnp.sqrt(var2 + 1e-5) * g2_ref[...] + bb2_ref[...]


def _full(shape):
    return pl.BlockSpec(shape, lambda b, t: (0,) * len(shape))


@jax.jit
def kernel(z, x, params):
    B = z.shape[0]
    f32 = jnp.float32

    zf = z.astype(f32)[..., None]
    xc = jnp.concatenate(
        [x, jnp.zeros((B, N, 5), f32)], axis=-1)

    tok = params["token_emb"]
    pos = params["pos_emb"]

    feats = pl.pallas_call(
        _emb_kernel,
        grid=(B, NT),
        in_specs=[
            pl.BlockSpec((1, T, 1), lambda b, t: (b, t, 0)),
            _full((NTOK, DIMF)),
            pl.BlockSpec((T, DIMF), lambda b, t: (t, 0)),
        ],
        out_specs=pl.BlockSpec((1, T, DIMF), lambda b, t: (b, t, 0)),
        out_shape=jax.ShapeDtypeStruct((B, N, DIMF), f32),
    )(zf, tok, pos)

    coors = xc
    for lp in params["layers"]:
        w1 = lp["edge1"]["w"]
        args = (
            jnp.transpose(coors, (0, 2, 1)),
            coors,
            feats,
            w1[:DIMF], w1[DIMF:2 * DIMF], w1[2 * DIMF:2 * DIMF + 1],
            lp["edge1"]["b"][None],
            lp["edge2"]["w"], lp["edge2"]["b"][None],
            lp["coor1"]["w"], lp["coor1"]["b"][None],
            lp["coor2"]["w"], lp["coor2"]["b"][None],
            lp["coors_scale"].reshape(1, 1),
            lp["node1"]["w"], lp["node1"]["b"][None],
            lp["node2"]["w"], lp["node2"]["b"][None],
        )
        feats, coors = pl.pallas_call(
            _layer_kernel,
            grid=(B, NT),
            in_specs=[
                pl.BlockSpec((1, 8, N), lambda b, t: (b, 0, 0)),
                pl.BlockSpec((1, N, 8), lambda b, t: (b, 0, 0)),
                pl.BlockSpec((1, N, DIMF), lambda b, t: (b, 0, 0)),
                _full((DIMF, H1)), _full((DIMF, H1)), _full((1, H1)),
                _full((1, H1)), _full((H1, MD)), _full((1, MD)),
                _full((MD, 4 * MD)), _full((1, 4 * MD)),
                _full((4 * MD, 1)), _full((1, 1)), _full((1, 1)),
                _full((DIMF + MD, 2 * DIMF)), _full((1, 2 * DIMF)),
                _full((2 * DIMF, DIMF)), _full((1, DIMF)),
            ],
            out_specs=[
                pl.BlockSpec((1, T, DIMF), lambda b, t: (b, t, 0)),
                pl.BlockSpec((1, T, 8), lambda b, t: (b, t, 0)),
            ],
            out_shape=[
                jax.ShapeDtypeStruct((B, N, DIMF), f32),
                jax.ShapeDtypeStruct((B, N, 8), f32),
            ],
            scratch_shapes=[
                pltpu.VMEM((T, N), f32),
                pltpu.VMEM((T, H1), f32),
                pltpu.VMEM((T, MD), f32),
                pltpu.VMEM((T, 8), f32),
            ],
        )(*args)

    h = pl.pallas_call(
        _ffn_kernel,
        grid=(B, NT),
        in_specs=[
            pl.BlockSpec((1, T, DIMF), lambda b, t: (b, t, 0)),
            _full((1, DIMF)), _full((1, DIMF)),
            _full((DIMF, 4 * DIMF)), _full((1, 4 * DIMF)),
            _full((4 * DIMF, DIMF)), _full((1, DIMF)),
            _full((1, DIMF)), _full((1, DIMF)),
        ],
        out_specs=pl.BlockSpec((1, T, DIMF), lambda b, t: (b, t, 0)),
        out_shape=jax.ShapeDtypeStruct((B, N, DIMF), f32),
    )(feats, params["norm1_g"][None], params["norm1_b"][None],
      params["ffn1"]["w"], params["ffn1"]["b"][None],
      params["ffn2"]["w"], params["ffn2"]["b"][None],
      params["norm2_g"][None], params["norm2_b"][None])

    return h, coors[..., :3]


# speedup vs baseline: 8.0744x; 2.5667x over previous
import jax
import jax.numpy as jnp
from jax.experimental import pallas as pl
from jax.experimental.pallas import tpu as pltpu

DIMF = 256
N = 1024
K = 32
MD = 16
NTOK = 21
T = 256
NT = N // T
H1 = (2 * DIMF + 1) * 2
CH = 8
NCH = K // CH
GW = DIMF + 24


def _silu(t):
    return t * jax.nn.sigmoid(t)


def _emb_kernel(zf_ref, tok_ref, pos_ref, out_ref):
    zf = zf_ref[0]
    iot = jax.lax.broadcasted_iota(jnp.int32, (1, NTOK), 1).astype(jnp.float32)
    oh = (zf == iot).astype(jnp.float32)
    out_ref[0] = (
        jnp.dot(oh, tok_ref[...], preferred_element_type=jnp.float32,
                precision=jax.lax.Precision.HIGHEST)
        + pos_ref[...]
    )


def _layer_kernel(
    xt_ref, gc_ref,
    wi_ref, wj_ref, wd_ref, b1_ref, w2_ref, b2_ref,
    wc1_ref, bc1_ref, wc2_ref, bc2_ref, sc_ref,
    wn1_ref, bn1_ref, wn2_ref, bn2_ref,
    fout_ref, cout_ref,
    d_ref, pi_ref, mi_ref, cd_ref, g_ref, v_ref,
):
    t = pl.program_id(1)
    fi = gc_ref[0, pl.ds(t * T, T), :DIMF]
    xia = gc_ref[0, pl.ds(t * T, T), DIMF:DIMF + 8]
    xib = gc_ref[0, pl.ds(t * T, T), DIMF + 8:DIMF + 16]
    xic = gc_ref[0, pl.ds(t * T, T), DIMF + 16:DIMF + 24]
    xi = (xia + xib) + xic
    xrow = xt_ref[0]

    d = None
    for c in range(3):
        r = xi[:, c:c + 1] - xrow[c:c + 1, :]
        r2 = r * r
        d = r2 if d is None else d + r2
    d_ref[...] = d

    pi_ref[...] = (
        jnp.dot(fi, wi_ref[...], preferred_element_type=jnp.float32)
        + b1_ref[...]
    )
    mi_ref[...] = jnp.zeros((T, MD), jnp.float32)
    cd_ref[...] = jnp.zeros((T, 8), jnp.float32)

    wdb = wd_ref[...].astype(jnp.bfloat16).astype(jnp.float32)

    for c4 in range(NCH):
        for rr in range(CH):
            dd = d_ref[...]
            v = jnp.min(dd, axis=1, keepdims=True)
            sel = dd == v
            d_ref[...] = jnp.where(sel, jnp.inf, dd)
            oh = sel.astype(jnp.float32)
            g_ref[pl.ds(rr * T, T), :] = jnp.dot(
                oh, gc_ref[0], preferred_element_type=jnp.float32)
            v_ref[pl.ds(rr * T, T), :] = v

        g2 = g_ref[:, :DIMF]
        xj2 = (g_ref[:, DIMF:DIMF + 8] + g_ref[:, DIMF + 8:DIMF + 16]) \
            + g_ref[:, DIMF + 16:DIMF + 24]
        vb2 = v_ref[...].astype(jnp.bfloat16).astype(jnp.float32)
        h1 = jnp.dot(g2, wj_ref[...], preferred_element_type=jnp.float32)
        h3 = (h1.reshape(CH, T, H1)
              + pi_ref[...][None, :, :]
              + vb2.reshape(CH, T, 1) * wdb[None, :, :])
        a1 = _silu(h3).reshape(CH * T, H1)
        m = _silu(jnp.dot(a1, w2_ref[...], preferred_element_type=jnp.float32)
                  + b2_ref[...])
        c1 = _silu(jnp.dot(m, wc1_ref[...], preferred_element_type=jnp.float32)
                   + bc1_ref[...])
        wgt = (jnp.dot(c1, wc2_ref[...], preferred_element_type=jnp.float32)
               + bc2_ref[...])
        rel = xi[None, :, :] - xj2.reshape(CH, T, 8)
        nrm = jnp.sqrt(jnp.sum(rel * rel, axis=-1, keepdims=True))
        reln = rel / jnp.maximum(nrm, 1e-8) * sc_ref[...][None]
        cd_ref[...] += jnp.sum(wgt.reshape(CH, T, 1) * reln, axis=0)
        mi_ref[...] += jnp.sum(m.reshape(CH, T, MD), axis=0)

    ni = jnp.concatenate([fi, mi_ref[...]], axis=-1)
    n1 = _silu(jnp.dot(ni, wn1_ref[...], preferred_element_type=jnp.float32)
               + bn1_ref[...])
    node = (jnp.dot(n1, wn2_ref[...], preferred_element_type=jnp.float32)
            + bn2_ref[...])
    fout_ref[0] = fi + node
    cout_ref[0] = xi + cd_ref[...]


def _ffn_kernel(f_ref, g1_ref, bb1_ref, wf1_ref, bf1_ref, wf2_ref, bf2_ref,
                g2_ref, bb2_ref, out_ref):
    h = f_ref[0]
    hh = h + h
    mu = jnp.mean(hh, axis=-1, keepdims=True)
    var = jnp.mean((hh - mu) ** 2, axis=-1, keepdims=True)
    hn = (hh - mu) / jnp.sqrt(var + 1e-5) * g1_ref[...] + bb1_ref[...]
    t1 = jax.nn.gelu(jnp.dot(hn, wf1_ref[...], preferred_element_type=jnp.float32)
                     + bf1_ref[...])
    h2 = (jnp.dot(t1, wf2_ref[...], preferred_element_type=jnp.float32)
          + bf2_ref[...])
    s = hn + h2
    mu2 = jnp.mean(s, axis=-1, keepdims=True)
    var2 = jnp.mean((s - mu2) ** 2, axis=-1, keepdims=True)
    out_ref[0] = (s - mu2) / jnp.sqrt(var2 + 1e-5) * g2_ref[...] + bb2_ref[...]


def _full(shape):
    return pl.BlockSpec(shape, lambda b, t: (0,) * len(shape))


def _limbs(xc):
    bf, f32 = jnp.bfloat16, jnp.float32
    xh = xc.astype(bf).astype(f32)
    xm = (xc - xh).astype(bf).astype(f32)
    xl = ((xc - xh) - xm).astype(bf).astype(f32)
    return xh, xm, xl


@jax.jit
def kernel(z, x, params):
    B = z.shape[0]
    f32 = jnp.float32

    zf = z.astype(f32)[..., None]
    xc = jnp.concatenate(
        [x, jnp.zeros((B, N, 5), f32)], axis=-1)

    feats = pl.pallas_call(
        _emb_kernel,
        grid=(B, NT),
        in_specs=[
            pl.BlockSpec((1, T, 1), lambda b, t: (b, t, 0)),
            _full((NTOK, DIMF)),
            pl.BlockSpec((T, DIMF), lambda b, t: (t, 0)),
        ],
        out_specs=pl.BlockSpec((1, T, DIMF), lambda b, t: (b, t, 0)),
        out_shape=jax.ShapeDtypeStruct((B, N, DIMF), f32),
    )(zf, params["token_emb"], params["pos_emb"])

    coors = xc
    for lp in params["layers"]:
        w1 = lp["edge1"]["w"]
        xh, xm, xl = _limbs(coors)
        gcat = jnp.concatenate([feats, xh, xm, xl], axis=-1)
        args = (
            jnp.transpose(coors, (0, 2, 1)),
            gcat,
            w1[:DIMF], w1[DIMF:2 * DIMF], w1[2 * DIMF:2 * DIMF + 1],
            lp["edge1"]["b"][None],
            lp["edge2"]["w"], lp["edge2"]["b"][None],
            lp["coor1"]["w"], lp["coor1"]["b"][None],
            lp["coor2"]["w"], lp["coor2"]["b"][None],
            lp["coors_scale"].reshape(1, 1),
            lp["node1"]["w"], lp["node1"]["b"][None],
            lp["node2"]["w"], lp["node2"]["b"][None],
        )
        feats, coors = pl.pallas_call(
            _layer_kernel,
            grid=(B, NT),
            in_specs=[
                pl.BlockSpec((1, 8, N), lambda b, t: (b, 0, 0)),
                pl.BlockSpec((1, N, GW), lambda b, t: (b, 0, 0)),
                _full((DIMF, H1)), _full((DIMF, H1)), _full((1, H1)),
                _full((1, H1)), _full((H1, MD)), _full((1, MD)),
                _full((MD, 4 * MD)), _full((1, 4 * MD)),
                _full((4 * MD, 1)), _full((1, 1)), _full((1, 1)),
                _full((DIMF + MD, 2 * DIMF)), _full((1, 2 * DIMF)),
                _full((2 * DIMF, DIMF)), _full((1, DIMF)),
            ],
            out_specs=[
                pl.BlockSpec((1, T, DIMF), lambda b, t: (b, t, 0)),
                pl.BlockSpec((1, T, 8), lambda b, t: (b, t, 0)),
            ],
            out_shape=[
                jax.ShapeDtypeStruct((B, N, DIMF), f32),
                jax.ShapeDtypeStruct((B, N, 8), f32),
            ],
            scratch_shapes=[
                pltpu.VMEM((T, N), f32),
                pltpu.VMEM((T, H1), f32),
                pltpu.VMEM((T, MD), f32),
                pltpu.VMEM((T, 8), f32),
                pltpu.VMEM((CH * T, GW), f32),
                pltpu.VMEM((CH * T, 1), f32),
            ],
        )(*args)

    h = pl.pallas_call(
        _ffn_kernel,
        grid=(B, NT),
        in_specs=[
            pl.BlockSpec((1, T, DIMF), lambda b, t: (b, t, 0)),
            _full((1, DIMF)), _full((1, DIMF)),
            _full((DIMF, 4 * DIMF)), _full((1, 4 * DIMF)),
            _full((4 * DIMF, DIMF)), _full((1, DIMF)),
            _full((1, DIMF)), _full((1, DIMF)),
        ],
        out_specs=pl.BlockSpec((1, T, DIMF), lambda b, t: (b, t, 0)),
        out_shape=jax.ShapeDtypeStruct((B, N, DIMF), f32),
    )(feats, params["norm1_g"][None], params["norm1_b"][None],
      params["ffn1"]["w"], params["ffn1"]["b"][None],
      params["ffn2"]["w"], params["ffn2"]["b"][None],
      params["norm2_g"][None], params["norm2_b"][None])

    return h, coors[..., :3]
